# baseline (speedup 1.0000x reference)
import functools

import jax
import jax.numpy as jnp
from jax.experimental import pallas as pl
from jax.experimental.pallas import tpu as pltpu
from jax.experimental.pallas import tpu_sc as plsc

NB = 512
CD = 64

_SC_INFO = plsc.get_sparse_core_info()
_NC = _SC_INFO.num_cores
_NS = _SC_INFO.num_subcores
_L = _SC_INFO.num_lanes
_NW = _NC * _NS


def _vq_body(x_ref, cb_ref, out_ref, idx_ref, loss_ref, cba_ref):
    first = (pl.program_id(0) == 0) & (pl.program_id(1) == 0)
    tt = x_ref.shape[2]

    @pl.when(first)
    def _():
        cb0 = cb_ref[...]
        cba_ref[:, :CD] = -2.0 * cb0
        cba_ref[:, CD:] = jnp.sum(cb0 * cb0, axis=1, keepdims=True)

    xt = x_ref[0]
    xn2 = jnp.sum(xt * xt, axis=0, keepdims=True)
    inv = jax.lax.rsqrt(jnp.maximum(xn2, 1e-24))
    xf = xt * inv
    xfn2 = xn2 * (inv * inv)
    xfa = jnp.concatenate([xf, jnp.ones((1, tt), jnp.float32)], axis=0)

    score = jax.lax.dot_general(cba_ref[...], xfa, (((1,), (0,)), ((), ())),
                                preferred_element_type=jnp.float32)

    idx = jnp.argmin(score, axis=0)
    idx_ref[0] = idx[None, :]
    onehot = (jax.lax.broadcasted_iota(jnp.int32, score.shape, 0)
              == idx[None, :]).astype(jnp.float32)

    xd = jax.lax.dot_general(cb_ref[...], onehot, (((0,), (0,)), ((), ())),
                             preferred_element_type=jnp.float32)
    out_ref[0] = xd

    mind = jnp.min(score, axis=0, keepdims=True) + xfn2
    lsum = jnp.sum(mind).reshape(1, 1)

    @pl.when(first)
    def _():
        loss_ref[...] = lsum

    @pl.when(jnp.logical_not(first))
    def _():
        loss_ref[...] = loss_ref[...] + lsum


def _hist_body(idx_hbm, out_hbm, idx_v, cnt_v):
    wid = jax.lax.axis_index("s") * _NC + jax.lax.axis_index("c")
    nper = idx_hbm.shape[0] // _NW
    base = wid * nper
    pltpu.sync_copy(idx_hbm.at[pl.ds(base, nper)], idx_v)

    def zero_row(b, c):
        cnt_v[pl.ds(b * _L, _L)] = jnp.zeros((_L,), jnp.float32)
        return c

    jax.lax.fori_loop(0, NB, zero_row, 0)

    lanes = jax.lax.iota(jnp.int32, _L)
    ones = jnp.ones((_L,), jnp.float32)

    def upd(c, carry):
        iv = idx_v[pl.ds(c * _L, _L)]
        plsc.addupdate_scatter(cnt_v, [iv * _L + lanes], ones)
        return carry

    jax.lax.fori_loop(0, nper // _L, upd, 0)
    pltpu.sync_copy(cnt_v, out_hbm.at[wid])


@functools.partial(jax.jit, static_argnames=("tt",))
def _vq(x, codebook, tt=2048):
    n, w, t = x.shape
    out, idx, lsum = pl.pallas_call(
        _vq_body,
        grid=(n, t // tt),
        in_specs=[
            pl.BlockSpec((1, w, tt), lambda i, j: (i, 0, j)),
            pl.BlockSpec((NB, CD), lambda i, j: (0, 0)),
        ],
        out_specs=[
            pl.BlockSpec((1, w, tt), lambda i, j: (i, 0, j)),
            pl.BlockSpec((1, 1, tt), lambda i, j: (i, 0, j)),
            pl.BlockSpec((1, 1), lambda i, j: (0, 0)),
        ],
        out_shape=[
            jax.ShapeDtypeStruct((n, w, t), jnp.float32),
            jax.ShapeDtypeStruct((n, 1, t), jnp.int32),
            jax.ShapeDtypeStruct((1, 1), jnp.float32),
        ],
        scratch_shapes=[pltpu.VMEM((NB, CD + 1), jnp.float32)],
    )(x, codebook)

    ntok = n * t
    hist = pl.kernel(
        _hist_body,
        mesh=plsc.VectorSubcoreMesh(core_axis_name="c", subcore_axis_name="s"),
        out_type=jax.ShapeDtypeStruct((_NW, NB * _L), jnp.float32),
        scratch_types=[
            pltpu.VMEM((ntok // _NW,), jnp.int32),
            pltpu.VMEM((NB * _L,), jnp.float32),
        ],
        compiler_params=pltpu.CompilerParams(needs_layout_passes=False),
    )
    parts = hist(idx.reshape(ntok))

    count = jnp.sum(parts.reshape(_NW, NB, _L), axis=(0, 2))
    prob = count / jnp.sum(count)
    perplexity = jnp.exp(-jnp.sum(prob * jnp.log(prob + 1e-7)))
    commit_loss = lsum[0, 0] / (ntok * w)
    return out, commit_loss, perplexity


def kernel(x, codebook):
    return _vq(x, codebook)


# speedup vs baseline: 1.2493x; 1.2493x over previous
import functools

import jax
import jax.numpy as jnp
from jax.experimental import pallas as pl
from jax.experimental.pallas import tpu as pltpu

NB = 512
CD = 64


def _vq_body(x_ref, cb_ref, out_ref, cnt_ref, loss_ref, cba_ref):
    first = pl.program_id(0) == 0
    nb_blk = x_ref.shape[0]
    tt = x_ref.shape[2]

    @pl.when(first)
    def _():
        cb0 = cb_ref[...]
        cba_ref[:, :CD] = -2.0 * cb0
        cba_ref[:, CD:] = jnp.sum(cb0 * cb0, axis=1, keepdims=True)

    cnt = jnp.zeros((NB, 1), jnp.float32)
    lsum = jnp.zeros((1, 1), jnp.float32)
    for b in range(nb_blk):
        xt = x_ref[b]
        xn2 = jnp.sum(xt * xt, axis=0, keepdims=True)
        inv = jax.lax.rsqrt(jnp.maximum(xn2, 1e-24))
        xf = xt * inv
        xfn2 = xn2 * (inv * inv)
        xfa = jnp.concatenate([xf, jnp.ones((1, tt), jnp.float32)], axis=0)

        score = jax.lax.dot_general(cba_ref[...], xfa,
                                    (((1,), (0,)), ((), ())),
                                    preferred_element_type=jnp.float32)

        idx = jnp.argmin(score, axis=0)
        onehot = (jax.lax.broadcasted_iota(jnp.int32, score.shape, 0)
                  == idx[None, :]).astype(jnp.float32)

        xd = jax.lax.dot_general(cb_ref[...], onehot,
                                 (((0,), (0,)), ((), ())),
                                 preferred_element_type=jnp.float32)
        out_ref[b] = xd

        mind = jnp.min(score, axis=0, keepdims=True) + xfn2
        cnt = cnt + jnp.sum(onehot, axis=1, keepdims=True)
        lsum = lsum + jnp.sum(mind).reshape(1, 1)

    @pl.when(first)
    def _():
        cnt_ref[...] = cnt
        loss_ref[...] = lsum

    @pl.when(jnp.logical_not(first))
    def _():
        cnt_ref[...] = cnt_ref[...] + cnt
        loss_ref[...] = loss_ref[...] + lsum


@functools.partial(jax.jit, static_argnames=("nb_blk", "tt"))
def _vq(x, codebook, nb_blk=2, tt=2048):
    n, w, t = x.shape
    out, cnt, lsum = pl.pallas_call(
        _vq_body,
        grid=(n // nb_blk,),
        in_specs=[
            pl.BlockSpec((nb_blk, w, tt), lambda i: (i, 0, 0)),
            pl.BlockSpec((NB, CD), lambda i: (0, 0)),
        ],
        out_specs=[
            pl.BlockSpec((nb_blk, w, tt), lambda i: (i, 0, 0)),
            pl.BlockSpec((NB, 1), lambda i: (0, 0)),
            pl.BlockSpec((1, 1), lambda i: (0, 0)),
        ],
        out_shape=[
            jax.ShapeDtypeStruct((n, w, t), jnp.float32),
            jax.ShapeDtypeStruct((NB, 1), jnp.float32),
            jax.ShapeDtypeStruct((1, 1), jnp.float32),
        ],
        scratch_shapes=[pltpu.VMEM((NB, CD + 1), jnp.float32)],
    )(x, codebook)
    ntok = n * t
    count = cnt[:, 0]
    prob = count / jnp.sum(count)
    perplexity = jnp.exp(-jnp.sum(prob * jnp.log(prob + 1e-7)))
    commit_loss = lsum[0, 0] / (ntok * w)
    return out, commit_loss, perplexity


def kernel(x, codebook):
    return _vq(x, codebook)


# speedup vs baseline: 1.2737x; 1.0196x over previous
import functools

import jax
import jax.numpy as jnp
from jax.experimental import pallas as pl
from jax.experimental.pallas import tpu as pltpu

NB = 512
CD = 64


def _vq_body(x_ref, cb_ref, out_ref, cnt_ref, loss_ref, cba_ref):
    first = pl.program_id(0) == 0
    nb_blk = x_ref.shape[0]
    tt = x_ref.shape[2]

    @pl.when(first)
    def _():
        cb0 = cb_ref[...]
        cba_ref[:, :CD] = -2.0 * cb0
        cba_ref[:, CD:] = jnp.sum(cb0 * cb0, axis=1, keepdims=True)

    cnt = jnp.zeros((NB, 1), jnp.float32)
    lsum = jnp.zeros((1, 1), jnp.float32)
    for b in range(nb_blk):
        xt = x_ref[b]
        xn2 = jnp.sum(xt * xt, axis=0, keepdims=True)
        inv = jax.lax.rsqrt(jnp.maximum(xn2, 1e-24))
        xf = xt * inv
        xfn2 = xn2 * (inv * inv)
        xfa = jnp.concatenate([xf, jnp.ones((1, tt), jnp.float32)], axis=0)

        score = jax.lax.dot_general(cba_ref[...], xfa,
                                    (((1,), (0,)), ((), ())),
                                    preferred_element_type=jnp.float32)

        idx = jnp.argmin(score, axis=0)
        onehot = (jax.lax.broadcasted_iota(jnp.int32, score.shape, 0)
                  == idx[None, :]).astype(jnp.float32)

        xd = jax.lax.dot_general(cb_ref[...], onehot,
                                 (((0,), (0,)), ((), ())),
                                 preferred_element_type=jnp.float32)
        out_ref[b] = xd

        mind = jnp.min(score, axis=0, keepdims=True) + xfn2
        cnt = cnt + jnp.sum(onehot, axis=1, keepdims=True)
        lsum = lsum + jnp.sum(mind).reshape(1, 1)

    @pl.when(first)
    def _():
        cnt_ref[...] = cnt
        loss_ref[...] = lsum

    @pl.when(jnp.logical_not(first))
    def _():
        cnt_ref[...] = cnt_ref[...] + cnt
        loss_ref[...] = loss_ref[...] + lsum


@functools.partial(jax.jit, static_argnames=("nb_blk", "tt"))
def _vq(x, codebook, nb_blk=4, tt=2048):
    n, w, t = x.shape
    out, cnt, lsum = pl.pallas_call(
        _vq_body,
        grid=(n // nb_blk,),
        in_specs=[
            pl.BlockSpec((nb_blk, w, tt), lambda i: (i, 0, 0)),
            pl.BlockSpec((NB, CD), lambda i: (0, 0)),
        ],
        out_specs=[
            pl.BlockSpec((nb_blk, w, tt), lambda i: (i, 0, 0)),
            pl.BlockSpec((NB, 1), lambda i: (0, 0)),
            pl.BlockSpec((1, 1), lambda i: (0, 0)),
        ],
        out_shape=[
            jax.ShapeDtypeStruct((n, w, t), jnp.float32),
            jax.ShapeDtypeStruct((NB, 1), jnp.float32),
            jax.ShapeDtypeStruct((1, 1), jnp.float32),
        ],
        scratch_shapes=[pltpu.VMEM((NB, CD + 1), jnp.float32)],
    )(x, codebook)
    ntok = n * t
    count = cnt[:, 0]
    prob = count / jnp.sum(count)
    perplexity = jnp.exp(-jnp.sum(prob * jnp.log(prob + 1e-7)))
    commit_loss = lsum[0, 0] / (ntok * w)
    return out, commit_loss, perplexity


def kernel(x, codebook):
    return _vq(x, codebook)
